# Initial kernel scaffold; baseline (speedup 1.0000x reference)
import functools

import jax
import jax.numpy as jnp
from jax import lax
from jax.experimental import pallas as pl
from jax.experimental.pallas import tpu as pltpu
from jax.experimental.pallas import tpu_sc as plsc

BATCH = 4096
HIST = 200
EMBED = 64
B = BATCH * HIST

_info = plsc.get_sparse_core_info()
NC, NS = _info.num_cores, _info.num_subcores
NW = NC * NS
BPW = B // NW
C = 128
NCH = BPW // C


def _body(x_hbm, table_hbm, out_hbm, idx_v, rows0, rows1, sem0, sem1):
    wid = lax.axis_index("s") * NC + lax.axis_index("c")
    base = wid * BPW

    pltpu.sync_copy(x_hbm.at[wid], idx_v)

    def start_gather(j, rows, sem):
        pltpu.async_copy(table_hbm.at[idx_v.at[j]], rows, sem)

    def wait_gather(rows, sem):
        pltpu.make_async_copy(table_hbm.at[pl.ds(0, C)], rows, sem).wait()

    def store(j, rows):
        off = pl.multiple_of(base + j * C, C)
        pltpu.sync_copy(rows, out_hbm.at[pl.ds(off, C)])

    start_gather(0, rows0, sem0)

    @pl.loop(0, NCH - 2, step=2)
    def _loop(i):
        start_gather(i + 1, rows1, sem1)
        wait_gather(rows0, sem0)
        store(i, rows0)
        start_gather(i + 2, rows0, sem0)
        wait_gather(rows1, sem1)
        store(i + 1, rows1)

    start_gather(NCH - 1, rows1, sem1)
    wait_gather(rows0, sem0)
    store(NCH - 2, rows0)
    wait_gather(rows1, sem1)
    store(NCH - 1, rows1)


_mesh = plsc.VectorSubcoreMesh(core_axis_name="c", subcore_axis_name="s")

_emb = functools.partial(
    pl.kernel,
    out_type=jax.ShapeDtypeStruct((B, EMBED), jnp.float32),
    mesh=_mesh,
    scratch_types=[
        pltpu.VMEM((NCH, C), jnp.int32),
        pltpu.VMEM((C, EMBED), jnp.float32),
        pltpu.VMEM((C, EMBED), jnp.float32),
        pltpu.SemaphoreType.DMA,
        pltpu.SemaphoreType.DMA,
    ],
)(_body)


def kernel(x, table):
    xr = x.reshape(NW, NCH, C).astype(jnp.int32)
    out = _emb(xr, table)
    return out.reshape(BATCH, HIST, EMBED)


# speedup vs baseline: 4.1172x; 4.1172x over previous
import functools

import jax
import jax.numpy as jnp
from jax import lax
from jax.experimental import pallas as pl
from jax.experimental.pallas import tpu as pltpu
from jax.experimental.pallas import tpu_sc as plsc

BATCH = 4096
HIST = 200
EMBED = 64
B = BATCH * HIST

_info = plsc.get_sparse_core_info()
NC, NS = _info.num_cores, _info.num_subcores
NW = NC * NS
BPW = B // NW
C = 128
NCH = BPW // C


def _body(x_hbm, table_hbm, out_hbm, idx_v, rows0, rows1, sem0, sem1):
    wid = lax.axis_index("s") * NC + lax.axis_index("c")
    base = wid * BPW

    pltpu.sync_copy(x_hbm.at[wid], idx_v)

    def start_gather(j, rows, sem):
        pltpu.async_copy(table_hbm.at[idx_v.at[j]], rows, sem)

    def wait_gather(rows, sem):
        pltpu.make_async_copy(table_hbm.at[pl.ds(0, C)], rows, sem).wait()

    def store(j, rows):
        off = pl.multiple_of(base + j * C, C)
        pltpu.sync_copy(rows, out_hbm.at[pl.ds(off, C)])

    start_gather(0, rows0, sem0)

    @pl.loop(0, NCH - 2, step=2)
    def _loop(i):
        start_gather(i + 1, rows1, sem1)
        wait_gather(rows0, sem0)
        store(i, rows0)
        start_gather(i + 2, rows0, sem0)
        wait_gather(rows1, sem1)
        store(i + 1, rows1)

    start_gather(NCH - 1, rows1, sem1)
    wait_gather(rows0, sem0)
    store(NCH - 2, rows0)
    wait_gather(rows1, sem1)
    store(NCH - 1, rows1)


_mesh = plsc.VectorSubcoreMesh(core_axis_name="c", subcore_axis_name="s")

_emb = functools.partial(
    pl.kernel,
    out_type=jax.ShapeDtypeStruct((B, EMBED), jnp.float32),
    mesh=_mesh,
    scratch_types=[
        pltpu.VMEM((NCH, C), jnp.int32),
        pltpu.VMEM((C, EMBED), jnp.float32),
        pltpu.VMEM((C, EMBED), jnp.float32),
        pltpu.SemaphoreType.DMA,
        pltpu.SemaphoreType.DMA,
    ],
    compiler_params=pltpu.CompilerParams(use_tc_tiling_on_sc=False),
)(_body)


def kernel(x, table):
    xr = x.reshape(NW, NCH, C).astype(jnp.int32)
    out = _emb(xr, table)
    return out.reshape(BATCH, HIST, EMBED)


# speedup vs baseline: 4.2592x; 1.0345x over previous
import functools

import jax
import jax.numpy as jnp
from jax import lax
from jax.experimental import pallas as pl
from jax.experimental.pallas import tpu as pltpu
from jax.experimental.pallas import tpu_sc as plsc

BATCH = 4096
HIST = 200
EMBED = 64
B = BATCH * HIST

_info = plsc.get_sparse_core_info()
NC, NS = _info.num_cores, _info.num_subcores
NW = NC * NS
BPW = B // NW
C = 128
NCH = BPW // C


NBUF = 8


def _body(x_hbm, table_hbm, out_hbm, idx_v, rows, *sems):
    sg, ss = sems[:NBUF], sems[NBUF:]
    wid = lax.axis_index("s") * NC + lax.axis_index("c")
    base = wid * BPW

    pltpu.sync_copy(x_hbm.at[wid], idx_v)

    def start_gather(j, b):
        pltpu.async_copy(table_hbm.at[idx_v.at[j]], rows.at[b], sg[b])

    def wait_gather(b):
        pltpu.make_async_copy(table_hbm.at[pl.ds(0, C)], rows.at[b], sg[b]).wait()

    def start_store(j, b):
        off = pl.multiple_of(base + j * C, C)
        pltpu.async_copy(rows.at[b], out_hbm.at[pl.ds(off, C)], ss[b])

    def wait_store(b):
        pltpu.make_async_copy(rows.at[b], out_hbm.at[pl.ds(0, C)], ss[b]).wait()

    for b in range(NBUF):
        start_gather(b, b)

    @pl.loop(0, NCH - NBUF, step=NBUF)
    def _loop(i):
        for b in range(NBUF):
            wait_gather(b)
            start_store(i + b, b)
        for b in range(NBUF):
            wait_store(b)
            start_gather(i + NBUF + b, b)

    i0 = NCH - NBUF
    for b in range(NBUF):
        wait_gather(b)
        start_store(i0 + b, b)
    for b in range(NBUF):
        wait_store(b)


_mesh = plsc.VectorSubcoreMesh(core_axis_name="c", subcore_axis_name="s")

_emb = functools.partial(
    pl.kernel,
    out_type=jax.ShapeDtypeStruct((B, EMBED), jnp.float32),
    mesh=_mesh,
    scratch_types=[
        pltpu.VMEM((NCH, C), jnp.int32),
        pltpu.VMEM((NBUF, C, EMBED), jnp.float32),
    ] + [pltpu.SemaphoreType.DMA] * (2 * NBUF),
    compiler_params=pltpu.CompilerParams(use_tc_tiling_on_sc=False),
)(_body)


def kernel(x, table):
    xr = x.reshape(NW, NCH, C).astype(jnp.int32)
    out = _emb(xr, table)
    return out.reshape(BATCH, HIST, EMBED)
